# baseline (speedup 1.0000x reference)
import functools

import jax
import jax.numpy as jnp
import numpy as _np
from jax import lax
from jax.experimental import pallas as pl
from jax.experimental.pallas import tpu as pltpu
from jax.experimental.pallas import tpu_sc as plsc

N = 10000
E = 160000
F = 128
T = 8
F_OUT = 4
HID = 32
N_GRAPHS = 64
EPS = 1e-5
TF = T * F

_DEG_HIST = _np.array([0.0] * 8 + [625.0] * 16 + [0.0] * 9)
_bins = _np.arange(_DEG_HIST.shape[0], dtype=_np.float64)
AVG_LOG = float((_np.log(_bins + 1.0) * _DEG_HIST).sum() / _DEG_HIST.sum())

NB = 400
P = 8
FC = 128
NW = 32
NPW = 320
NH = 160
CHUNK = 128
QCAP = 3456
SCHUNK = 1024
GCH = 128
EPAD = NW * NPW
BIG = 1e30


def _cs_body(x_ref, w1_ref, w2_ref, c0_ref, c_ref, s_ref):
    xb = x_ref[...]
    c_ref[0] = jnp.dot(xb, w1_ref[0], preferred_element_type=jnp.float32) + c0_ref[0, 0:1, :]
    s_ref[0] = jnp.dot(xb, w2_ref[0], preferred_element_type=jnp.float32)


def _compute_c_s(x, w13, w23, c03):
    return pl.pallas_call(
        _cs_body,
        grid=(N // NB, P),
        in_specs=[
            pl.BlockSpec((NB, F), lambda i, p: (i, 0)),
            pl.BlockSpec((1, F, FC), lambda i, p: (p, 0, 0)),
            pl.BlockSpec((1, F, FC), lambda i, p: (p, 0, 0)),
            pl.BlockSpec((1, 8, FC), lambda i, p: (p, 0, 0)),
        ],
        out_specs=[
            pl.BlockSpec((1, NB, FC), lambda i, p: (p, i, 0)),
            pl.BlockSpec((1, NB, FC), lambda i, p: (p, i, 0)),
        ],
        out_shape=[
            jax.ShapeDtypeStruct((P, N, FC), jnp.float32),
            jax.ShapeDtypeStruct((P, N, FC), jnp.float32),
        ],
    )(x, w13, w23, c03)


def _sc_edge_body(s2d, rec, v3, q_out, deg_out,
                  acc, dacc, qsrc, qdst, qea, recb, rows, gidx, vvb, sem):
    wid = lax.axis_index("c") * 16 + lax.axis_index("s")
    n0w = pl.multiple_of(wid * NPW, 16)

    def qz_body(i, _):
        z = jnp.zeros((16,), jnp.int32)
        qsrc[pl.ds(i * 16, 16)] = z
        qdst[pl.ds(i * 16, 16)] = jnp.full((16,), NH, jnp.int32)
        qea[pl.ds(i * 16, 16)] = z
        return 0
    lax.fori_loop(0, 2 * QCAP // 16, qz_body, 0)

    def scan_chunk(c, curs):
        base = pl.multiple_of(c * SCHUNK, 8)
        pltpu.sync_copy(rec.at[:, pl.ds(base, SCHUNK)], recb)

        def scan_vec(i, curs):
            c_lo, c_hi = curs
            sv = recb[0, pl.ds(i * 16, 16)]
            dv = recb[1, pl.ds(i * 16, 16)]
            ev = recb[2, pl.ds(i * 16, 16)]
            dl = dv - n0w
            m_lo = (dl >= 0) & (dl < NH)
            m_hi = (dl >= NH) & (dl < NPW)
            pos_lo = plsc.cumsum(m_lo.astype(jnp.int32))
            pos_hi = plsc.cumsum(m_hi.astype(jnp.int32))
            trash = jnp.full((16,), 2 * QCAP + 15, jnp.int32)
            idx_lo = jnp.where(m_lo, c_lo + pos_lo - 1, trash)
            idx_hi = jnp.where(m_hi, QCAP + c_hi + pos_hi - 1, trash)
            plsc.store_scatter(qsrc, [idx_lo], sv)
            plsc.store_scatter(qdst, [idx_lo], dl)
            plsc.store_scatter(qea, [idx_lo], ev)
            plsc.store_scatter(qsrc, [idx_hi], sv)
            plsc.store_scatter(qdst, [idx_hi], dl - NH)
            plsc.store_scatter(qea, [idx_hi], ev)
            c_lo = jnp.minimum(c_lo + pos_lo[15], QCAP - 16)
            c_hi = jnp.minimum(c_hi + pos_hi[15], QCAP - 16)
            return (c_lo, c_hi)

        return lax.fori_loop(0, SCHUNK // 16, scan_vec, curs)

    cnt_lo, cnt_hi = lax.fori_loop(0, E // SCHUNK, scan_chunk,
                                   (jnp.int32(0), jnp.int32(0)))
    padv_s = jnp.zeros((16,), jnp.int32)
    padv_d = jnp.full((16,), NH, jnp.int32)
    qsrc[pl.ds(cnt_lo, 16)] = padv_s
    qdst[pl.ds(cnt_lo, 16)] = padv_d
    qea[pl.ds(cnt_lo, 16)] = padv_s
    qsrc[pl.ds(QCAP + cnt_hi, 16)] = padv_s
    qdst[pl.ds(QCAP + cnt_hi, 16)] = padv_d
    qea[pl.ds(QCAP + cnt_hi, 16)] = padv_s
    ng_lo = lax.div(cnt_lo + 15, 16)
    ng_hi = lax.div(cnt_hi + 15, 16)

    zero = jnp.zeros((16,), jnp.float32)
    big = jnp.full((16,), BIG, jnp.float32)
    nbig = jnp.full((16,), -BIG, jnp.float32)
    one0 = (lax.iota(jnp.int32, 16) == 0).astype(jnp.float32)

    def pass_body(p, _):
        pltpu.sync_copy(v3.at[p], vvb)
        vv = [vvb[pl.ds(k * 16, 16)] for k in range(8)]
        pN = p * N

        def half_body(h, _):
            qb = h * QCAP
            ngrp = jnp.where(h == 0, ng_lo, ng_hi)
            n0 = n0w + h * NH

            def init_body(r, _):
                rb = r * (4 * FC)
                for k in range(8):
                    acc[pl.ds(rb + k * 16, 16)] = zero
                for k in range(8):
                    acc[pl.ds(rb + 128 + k * 16, 16)] = zero
                for k in range(8):
                    acc[pl.ds(rb + 256 + k * 16, 16)] = big
                for k in range(8):
                    acc[pl.ds(rb + 384 + k * 16, 16)] = nbig
                dacc[pl.ds(r * 16, 16)] = zero
                return 0
            lax.fori_loop(0, NH + 1, init_body, 0)

            def gchunk_body(c, _):
                gb = qb + c * GCH
                for i in range(GCH // 16):
                    gidx[pl.ds(i * 16, 16)] = qsrc[pl.ds(gb + i * 16, 16)] + pN
                pltpu.async_copy(s2d.at[gidx], rows, sem).wait()

                def grp_body(g, _):
                    dlv = qdst[pl.ds(gb + g * 16, 16)]
                    eav = plsc.bitcast(qea[pl.ds(gb + g * 16, 16)], jnp.float32)
                    for l in range(16):
                        loc = dlv[l]
                        ea = eav[l]
                        sb = loc * (4 * FC)
                        db = loc * 16
                        plsc.addupdate(dacc.at[pl.ds(db, 16)], one0)
                        for k in range(8):
                            rowv = rows[g * 16 + l, pl.ds(k * 16, 16)]
                            qv = rowv + ea * vv[k]
                            plsc.addupdate(acc.at[pl.ds(sb + k * 16, 16)], qv)
                            plsc.addupdate(acc.at[pl.ds(sb + 128 + k * 16, 16)], qv * qv)
                            o2 = sb + 256 + k * 16
                            acc[pl.ds(o2, 16)] = jnp.minimum(acc[pl.ds(o2, 16)], qv)
                            o3 = sb + 384 + k * 16
                            acc[pl.ds(o3, 16)] = jnp.maximum(acc[pl.ds(o3, 16)], qv)
                    return 0

                ngl = jnp.minimum(ngrp - c * (GCH // 16), GCH // 16)
                ngl = jnp.maximum(ngl, 0)
                lax.fori_loop(0, ngl, grp_body, 0)
                return 0

            nchunk = lax.div(ngrp + (GCH // 16 - 1), GCH // 16)
            lax.fori_loop(0, nchunk, gchunk_body, 0)

            pltpu.sync_copy(acc.at[pl.ds(0, NH * 4 * FC)],
                            q_out.at[p, pl.ds(n0 * (4 * FC), NH * 4 * FC)])

            @pl.when(p == 0)
            def _deg_out():
                pltpu.sync_copy(dacc.at[pl.ds(0, NH * 16)],
                                deg_out.at[pl.ds(n0 * 16, NH * 16)])
            return 0

        lax.fori_loop(0, 2, half_body, 0)
        return 0

    lax.fori_loop(0, P, pass_body, 0)


def _sc_edge(s2d, rec, v3):
    mesh = plsc.VectorSubcoreMesh(core_axis_name="c", subcore_axis_name="s")
    f = pl.kernel(
        _sc_edge_body,
        out_type=[
            jax.ShapeDtypeStruct((P, EPAD * 4 * FC), jnp.float32),
            jax.ShapeDtypeStruct((EPAD * 16,), jnp.float32),
        ],
        mesh=mesh,
        compiler_params=pltpu.CompilerParams(needs_layout_passes=False),
        scratch_types=[
            pltpu.VMEM(((NH + 1) * 4 * FC,), jnp.float32),
            pltpu.VMEM(((NH + 1) * 16,), jnp.float32),
            pltpu.VMEM((2 * QCAP + 16,), jnp.int32),
            pltpu.VMEM((2 * QCAP + 16,), jnp.int32),
            pltpu.VMEM((2 * QCAP + 16,), jnp.int32),
            pltpu.VMEM((3, SCHUNK), jnp.int32),
            pltpu.VMEM((GCH, FC), jnp.float32),
            pltpu.VMEM((GCH,), jnp.int32),
            pltpu.VMEM((FC,), jnp.float32),
            pltpu.SemaphoreType.DMA,
        ],
    )
    return f(s2d, rec, v3)


def _post_body(x_ref, c_ref, q_ref, deg_ref, wx_ref, pw_ref, pb_ref,
               lw_ref, lb_ref, o_ref, s1_ref, s2_ref, acc_ref):
    i = pl.program_id(0)
    p = pl.program_id(1)
    deg = deg_ref[:, 0:1]
    degc = jnp.maximum(deg, 1.0)
    has = deg > 0
    C = c_ref[0]
    qblk = q_ref[0]
    Q1 = jnp.where(has, qblk[:, 0:FC], 0.0)
    Q2 = jnp.where(has, qblk[:, FC:2 * FC], 0.0)
    Qmn = jnp.where(has, qblk[:, 2 * FC:3 * FC], 0.0)
    Qmx = jnp.where(has, qblk[:, 3 * FC:4 * FC], 0.0)
    mean = (deg * C + Q1) / degc
    e2 = (deg * C * C + 2.0 * C * Q1 + Q2) / degc
    std = jnp.sqrt(jax.nn.relu(e2 - mean * mean) + EPS)
    mn = jnp.where(has, C + Qmn, 0.0)
    mx = jnp.where(has, C + Qmx, 0.0)
    amp = jnp.log(deg + 1.0) / AVG_LOG
    att = AVG_LOG / jnp.log(degc + 1.0)

    @pl.when(p == 0)
    def _init_acc():
        acc_ref[...] = jnp.dot(x_ref[...], wx_ref[...],
                               preferred_element_type=jnp.float32) + pb_ref[...]

    acc = acc_ref[...]
    parts = (mean, mn, mx, std)
    for k in range(4):
        a = parts[k]
        yp = jnp.dot(a, pw_ref[0, k], preferred_element_type=jnp.float32)
        ya = jnp.dot(a, pw_ref[0, 4 + k], preferred_element_type=jnp.float32)
        yt = jnp.dot(a, pw_ref[0, 8 + k], preferred_element_type=jnp.float32)
        acc += yp + amp * ya + att * yt
    acc_ref[...] = acc

    @pl.when(p == P - 1)
    def _fin():
        out = jnp.dot(acc, lw_ref[...], preferred_element_type=jnp.float32) + lb_ref[...]
        o_ref[...] = out

        @pl.when(i == 0)
        def _init_s():
            s1_ref[...] = jnp.zeros_like(s1_ref)
            s2_ref[...] = jnp.zeros_like(s2_ref)

        s1_ref[...] += jnp.sum(out, axis=0, keepdims=True)
        s2_ref[...] += jnp.sum(out * out, axis=0, keepdims=True)


def _post(x, C3, Q, deg2d, wx, pw, pb, lin_W, lin_b):
    return pl.pallas_call(
        _post_body,
        grid=(N // NB, P),
        in_specs=[
            pl.BlockSpec((NB, F), lambda i, p: (i, 0)),
            pl.BlockSpec((1, NB, FC), lambda i, p: (p, i, 0)),
            pl.BlockSpec((1, NB, 4 * FC), lambda i, p: (p, i, 0)),
            pl.BlockSpec((NB, 16), lambda i, p: (i, 0)),
            pl.BlockSpec((F, HID), lambda i, p: (0, 0)),
            pl.BlockSpec((1, 12, F, HID), lambda i, p: (p, 0, 0, 0)),
            pl.BlockSpec((1, HID), lambda i, p: (0, 0)),
            pl.BlockSpec((HID, HID), lambda i, p: (0, 0)),
            pl.BlockSpec((1, HID), lambda i, p: (0, 0)),
        ],
        out_specs=[
            pl.BlockSpec((NB, HID), lambda i, p: (i, 0)),
            pl.BlockSpec((1, HID), lambda i, p: (0, 0)),
            pl.BlockSpec((1, HID), lambda i, p: (0, 0)),
        ],
        out_shape=[
            jax.ShapeDtypeStruct((N, HID), jnp.float32),
            jax.ShapeDtypeStruct((1, HID), jnp.float32),
            jax.ShapeDtypeStruct((1, HID), jnp.float32),
        ],
        scratch_shapes=[pltpu.VMEM((NB, HID), jnp.float32)],
    )(x, C3, Q, deg2d, wx, pw, pb, lin_W, lin_b)


PAD = 8


def _final_body(o_ref, s1_ref, s2_ref, oneh_ref, bg_ref, bb_ref,
                w1_ref, b1_ref, w2_ref, b2_ref, w3_ref, b3_ref, out_ref,
                acc_ref):
    i = pl.program_id(0)
    mu = s1_ref[...] / N
    var = s2_ref[...] / N - mu * mu
    o = (o_ref[...] - mu) / jnp.sqrt(var + EPS) * bg_ref[...] + bb_ref[...]
    o = jax.nn.relu(o)

    @pl.when(i == 0)
    def _init():
        acc_ref[...] = jnp.zeros_like(acc_ref)

    acc_ref[...] += jax.lax.dot_general(
        oneh_ref[...], o, (((0,), (0,)), ((), ())),
        preferred_element_type=jnp.float32)

    @pl.when(i == pl.num_programs(0) - 1)
    def _fin():
        g = acc_ref[...]
        g = jax.nn.relu(jnp.dot(g, w1_ref[...], preferred_element_type=jnp.float32) + b1_ref[...])
        g = jax.nn.relu(jnp.dot(g, w2_ref[...], preferred_element_type=jnp.float32) + b2_ref[...])
        g = jnp.dot(g, w3_ref[...], preferred_element_type=jnp.float32) + b3_ref[...]
        col = jax.lax.broadcasted_iota(jnp.int32, (N_GRAPHS, PAD), 1)
        g = jnp.where(col < 2, g, -1e30)
        m = jnp.max(g, axis=-1, keepdims=True)
        s = jnp.log(jnp.sum(jnp.exp(g - m), axis=-1, keepdims=True))
        out_ref[...] = g - m - s


def _final(o, s1, s2, oneh, bn_gamma, bn_beta, mlp_W1, mlp_b1, mlp_W2, mlp_b2,
           w3p, b3p):
    return pl.pallas_call(
        _final_body,
        grid=(N // NB,),
        in_specs=[
            pl.BlockSpec((NB, HID), lambda i: (i, 0)),
            pl.BlockSpec((1, HID), lambda i: (0, 0)),
            pl.BlockSpec((1, HID), lambda i: (0, 0)),
            pl.BlockSpec((NB, N_GRAPHS), lambda i: (i, 0)),
            pl.BlockSpec((1, HID), lambda i: (0, 0)),
            pl.BlockSpec((1, HID), lambda i: (0, 0)),
            pl.BlockSpec((HID, HID // 2), lambda i: (0, 0)),
            pl.BlockSpec((1, HID // 2), lambda i: (0, 0)),
            pl.BlockSpec((HID // 2, HID // 4), lambda i: (0, 0)),
            pl.BlockSpec((1, HID // 4), lambda i: (0, 0)),
            pl.BlockSpec((HID // 4, PAD), lambda i: (0, 0)),
            pl.BlockSpec((1, PAD), lambda i: (0, 0)),
        ],
        out_specs=pl.BlockSpec((N_GRAPHS, PAD), lambda i: (0, 0)),
        out_shape=jax.ShapeDtypeStruct((N_GRAPHS, PAD), jnp.float32),
        scratch_shapes=[pltpu.VMEM((N_GRAPHS, HID), jnp.float32)],
    )(o, s1, s2, oneh, bn_gamma, bn_beta, mlp_W1, mlp_b1, mlp_W2, mlp_b2,
      w3p, b3p)


def kernel(x, edge_index, edge_attr, batch, edge_enc_W, edge_enc_b, pre_W,
           pre_b, post_W, post_b, lin_W, lin_b, bn_gamma, bn_beta, mlp_W1,
           mlp_b1, mlp_W2, mlp_b2, mlp_W3, mlp_b3):
    src, dst = edge_index[0], edge_index[1]
    W1 = pre_W[:, :F, :]
    W2 = pre_W[:, F:2 * F, :]
    W3 = pre_W[:, 2 * F:, :]
    v = jnp.einsum('f,tfo->to', edge_enc_W[0], W3).reshape(-1)
    c0 = (jnp.einsum('f,tfo->to', edge_enc_b, W3) + pre_b).reshape(-1)

    w13 = W1.transpose(1, 0, 2).reshape(F, TF).reshape(F, P, FC).transpose(1, 0, 2)
    w23 = W2.transpose(1, 0, 2).reshape(F, TF).reshape(F, P, FC).transpose(1, 0, 2)
    c03 = jnp.broadcast_to(c0.reshape(P, 1, FC), (P, 8, FC))
    C3, S3 = _compute_c_s(x, w13, w23, c03)
    s2d = S3.reshape(P * N, FC)

    rec = jnp.stack([src.astype(jnp.int32), dst.astype(jnp.int32),
                     jax.lax.bitcast_convert_type(edge_attr, jnp.int32)])
    v3 = v.reshape(P, FC)

    Qf, degq = _sc_edge(s2d, rec, v3)
    Q = Qf.reshape(P, EPAD, 4 * FC)
    deg2d = degq.reshape(EPAD, 16)

    wx = post_W[:, :F, :].transpose(1, 0, 2).reshape(F, T * F_OUT)
    eye_t = jnp.eye(T, dtype=jnp.float32)
    pw_parts = []
    for k in range(12):
        wk = post_W[:, (1 + k) * F:(2 + k) * F, :]
        pw_parts.append(jnp.einsum('pfo,pt->pfto', wk, eye_t).reshape(P, F, T * F_OUT))
    pw = jnp.stack(pw_parts, axis=1)
    pb = post_b.reshape(1, HID)

    o, s1, s2 = _post(x, C3, Q, deg2d, wx, pw, pb, lin_W,
                      lin_b.reshape(1, -1))

    oneh = (batch[:, None] == jnp.arange(N_GRAPHS, dtype=batch.dtype)[None, :]).astype(jnp.float32)
    w3p = jnp.zeros((HID // 4, PAD), jnp.float32).at[:, :2].set(mlp_W3)
    b3p = jnp.zeros((1, PAD), jnp.float32).at[:, :2].set(mlp_b3)
    outp = _final(o, s1, s2, oneh, bn_gamma.reshape(1, HID),
                  bn_beta.reshape(1, HID), mlp_W1, mlp_b1.reshape(1, -1),
                  mlp_W2, mlp_b2.reshape(1, -1), w3p, b3p)
    return outp[:, :2]


# speedup vs baseline: 1.1125x; 1.1125x over previous
import functools

import jax
import jax.numpy as jnp
import numpy as _np
from jax import lax
from jax.experimental import pallas as pl
from jax.experimental.pallas import tpu as pltpu
from jax.experimental.pallas import tpu_sc as plsc

N = 10000
E = 160000
F = 128
T = 8
F_OUT = 4
HID = 32
N_GRAPHS = 64
EPS = 1e-5
TF = T * F

_DEG_HIST = _np.array([0.0] * 8 + [625.0] * 16 + [0.0] * 9)
_bins = _np.arange(_DEG_HIST.shape[0], dtype=_np.float64)
AVG_LOG = float((_np.log(_bins + 1.0) * _DEG_HIST).sum() / _DEG_HIST.sum())

NB = 400
P = 8
FC = 128
NW = 32
NPW = 320
NH = 160
CHUNK = 128
QCAP = 3456
SCHUNK = 1024
GCH = 64
EPAD = NW * NPW
BIG = 1e30


def _cs_body(x_ref, w1_ref, w2_ref, c0_ref, c_ref, s_ref):
    xb = x_ref[...]
    c_ref[0] = jnp.dot(xb, w1_ref[0], preferred_element_type=jnp.float32) + c0_ref[0, 0:1, :]
    s_ref[0] = jnp.dot(xb, w2_ref[0], preferred_element_type=jnp.float32)


def _compute_c_s(x, w13, w23, c03):
    return pl.pallas_call(
        _cs_body,
        grid=(N // NB, P),
        in_specs=[
            pl.BlockSpec((NB, F), lambda i, p: (i, 0)),
            pl.BlockSpec((1, F, FC), lambda i, p: (p, 0, 0)),
            pl.BlockSpec((1, F, FC), lambda i, p: (p, 0, 0)),
            pl.BlockSpec((1, 8, FC), lambda i, p: (p, 0, 0)),
        ],
        out_specs=[
            pl.BlockSpec((1, NB, FC), lambda i, p: (p, i, 0)),
            pl.BlockSpec((1, NB, FC), lambda i, p: (p, i, 0)),
        ],
        out_shape=[
            jax.ShapeDtypeStruct((P, N, FC), jnp.float32),
            jax.ShapeDtypeStruct((P, N, FC), jnp.float32),
        ],
    )(x, w13, w23, c03)


def _sc_edge_body(s2d, rec, v3, q_out, deg_out,
                  acc, dacc, qsrc, qdst, qea, recb, rows, gidx, vvb, sem):
    wid = lax.axis_index("c") * 16 + lax.axis_index("s")
    n0w = pl.multiple_of(wid * NPW, 16)

    def qz_body(i, _):
        z = jnp.zeros((16,), jnp.int32)
        qsrc[pl.ds(i * 16, 16)] = z
        qdst[pl.ds(i * 16, 16)] = jnp.full((16,), NH, jnp.int32)
        qea[pl.ds(i * 16, 16)] = z
        return 0
    lax.fori_loop(0, 2 * QCAP // 16, qz_body, 0)

    def scan_chunk(c, curs):
        base = pl.multiple_of(c * SCHUNK, 8)
        pltpu.sync_copy(rec.at[:, pl.ds(base, SCHUNK)], recb)

        def scan_vec(i, curs):
            c_lo, c_hi = curs
            sv = recb[0, pl.ds(i * 16, 16)]
            dv = recb[1, pl.ds(i * 16, 16)]
            ev = recb[2, pl.ds(i * 16, 16)]
            dl = dv - n0w
            m_lo = (dl >= 0) & (dl < NH)
            m_hi = (dl >= NH) & (dl < NPW)
            pos_lo = plsc.cumsum(m_lo.astype(jnp.int32))
            pos_hi = plsc.cumsum(m_hi.astype(jnp.int32))
            trash = jnp.full((16,), 2 * QCAP + 15, jnp.int32)
            idx_lo = jnp.where(m_lo, c_lo + pos_lo - 1, trash)
            idx_hi = jnp.where(m_hi, QCAP + c_hi + pos_hi - 1, trash)
            plsc.store_scatter(qsrc, [idx_lo], sv)
            plsc.store_scatter(qdst, [idx_lo], dl)
            plsc.store_scatter(qea, [idx_lo], ev)
            plsc.store_scatter(qsrc, [idx_hi], sv)
            plsc.store_scatter(qdst, [idx_hi], dl - NH)
            plsc.store_scatter(qea, [idx_hi], ev)
            c_lo = jnp.minimum(c_lo + pos_lo[15], QCAP - 16)
            c_hi = jnp.minimum(c_hi + pos_hi[15], QCAP - 16)
            return (c_lo, c_hi)

        return lax.fori_loop(0, SCHUNK // 16, scan_vec, curs)

    cnt_lo, cnt_hi = lax.fori_loop(0, E // SCHUNK, scan_chunk,
                                   (jnp.int32(0), jnp.int32(0)))
    padv_s = jnp.zeros((16,), jnp.int32)
    padv_d = jnp.full((16,), NH, jnp.int32)
    qsrc[pl.ds(cnt_lo, 16)] = padv_s
    qdst[pl.ds(cnt_lo, 16)] = padv_d
    qea[pl.ds(cnt_lo, 16)] = padv_s
    qsrc[pl.ds(QCAP + cnt_hi, 16)] = padv_s
    qdst[pl.ds(QCAP + cnt_hi, 16)] = padv_d
    qea[pl.ds(QCAP + cnt_hi, 16)] = padv_s
    ng_lo = lax.div(cnt_lo + 15, 16)
    ng_hi = lax.div(cnt_hi + 15, 16)

    zero = jnp.zeros((16,), jnp.float32)
    big = jnp.full((16,), BIG, jnp.float32)
    nbig = jnp.full((16,), -BIG, jnp.float32)
    one0 = (lax.iota(jnp.int32, 16) == 0).astype(jnp.float32)

    def pass_body(p, _):
        pltpu.sync_copy(v3.at[p], vvb)
        vv = [vvb[pl.ds(k * 16, 16)] for k in range(8)]
        pN = p * N

        def half_body(h, _):
            qb = h * QCAP
            ngrp = jnp.where(h == 0, ng_lo, ng_hi)
            n0 = n0w + h * NH

            def init_body(r, _):
                rb = r * (4 * FC)
                for k in range(8):
                    acc[pl.ds(rb + k * 16, 16)] = zero
                for k in range(8):
                    acc[pl.ds(rb + 128 + k * 16, 16)] = zero
                for k in range(8):
                    acc[pl.ds(rb + 256 + k * 16, 16)] = big
                for k in range(8):
                    acc[pl.ds(rb + 384 + k * 16, 16)] = nbig
                dacc[pl.ds(r * 16, 16)] = zero
                return 0
            lax.fori_loop(0, NH + 1, init_body, 0)

            nchunk = lax.div(ngrp + (GCH // 16 - 1), GCH // 16)

            def issue(c, par):
                gb = qb + c * GCH
                for i in range(GCH // 16):
                    gidx[par, pl.ds(i * 16, 16)] = (
                        qsrc[pl.ds(gb + i * 16, 16)] + pN)
                pltpu.async_copy(s2d.at[gidx.at[par]], rows.at[par],
                                 sem.at[par])

            @pl.when(nchunk > 0)
            def _prime():
                issue(jnp.int32(0), jnp.int32(0))

            def gchunk_body(c, _):
                par = lax.rem(c, 2)
                gb = qb + c * GCH
                pltpu.make_async_copy(s2d.at[gidx.at[par]], rows.at[par],
                                      sem.at[par]).wait()

                @pl.when(c + 1 < nchunk)
                def _next():
                    issue(c + 1, 1 - par)

                def grp_body(g, _):
                    dlv = qdst[pl.ds(gb + g * 16, 16)]
                    eav = plsc.bitcast(qea[pl.ds(gb + g * 16, 16)], jnp.float32)
                    for l in range(16):
                        loc = dlv[l]
                        ea = eav[l]
                        sb = loc * (4 * FC)
                        db = loc * 16
                        plsc.addupdate(dacc.at[pl.ds(db, 16)], one0)
                        for k in range(8):
                            rowv = rows[par, g * 16 + l, pl.ds(k * 16, 16)]
                            qv = rowv + ea * vv[k]
                            plsc.addupdate(acc.at[pl.ds(sb + k * 16, 16)], qv)
                            plsc.addupdate(acc.at[pl.ds(sb + 128 + k * 16, 16)], qv * qv)
                            o2 = sb + 256 + k * 16
                            acc[pl.ds(o2, 16)] = jnp.minimum(acc[pl.ds(o2, 16)], qv)
                            o3 = sb + 384 + k * 16
                            acc[pl.ds(o3, 16)] = jnp.maximum(acc[pl.ds(o3, 16)], qv)
                    return 0

                ngl = jnp.minimum(ngrp - c * (GCH // 16), GCH // 16)
                ngl = jnp.maximum(ngl, 0)
                lax.fori_loop(0, ngl, grp_body, 0)
                return 0

            lax.fori_loop(0, nchunk, gchunk_body, 0)

            pltpu.sync_copy(acc.at[pl.ds(0, NH * 4 * FC)],
                            q_out.at[p, pl.ds(n0 * (4 * FC), NH * 4 * FC)])

            @pl.when(p == 0)
            def _deg_out():
                pltpu.sync_copy(dacc.at[pl.ds(0, NH * 16)],
                                deg_out.at[pl.ds(n0 * 16, NH * 16)])
            return 0

        lax.fori_loop(0, 2, half_body, 0)
        return 0

    lax.fori_loop(0, P, pass_body, 0)


def _sc_edge(s2d, rec, v3):
    mesh = plsc.VectorSubcoreMesh(core_axis_name="c", subcore_axis_name="s")
    f = pl.kernel(
        _sc_edge_body,
        out_type=[
            jax.ShapeDtypeStruct((P, EPAD * 4 * FC), jnp.float32),
            jax.ShapeDtypeStruct((EPAD * 16,), jnp.float32),
        ],
        mesh=mesh,
        compiler_params=pltpu.CompilerParams(needs_layout_passes=False),
        scratch_types=[
            pltpu.VMEM(((NH + 1) * 4 * FC,), jnp.float32),
            pltpu.VMEM(((NH + 1) * 16,), jnp.float32),
            pltpu.VMEM((2 * QCAP + 16,), jnp.int32),
            pltpu.VMEM((2 * QCAP + 16,), jnp.int32),
            pltpu.VMEM((2 * QCAP + 16,), jnp.int32),
            pltpu.VMEM((3, SCHUNK), jnp.int32),
            pltpu.VMEM((2, GCH, FC), jnp.float32),
            pltpu.VMEM((2, GCH), jnp.int32),
            pltpu.VMEM((FC,), jnp.float32),
            pltpu.SemaphoreType.DMA((2,)),
        ],
    )
    return f(s2d, rec, v3)


def _post_body(x_ref, c_ref, q_ref, deg_ref, wx_ref, pw_ref, pb_ref,
               lw_ref, lb_ref, o_ref, s1_ref, s2_ref, acc_ref):
    i = pl.program_id(0)
    p = pl.program_id(1)
    deg = deg_ref[:, 0:1]
    degc = jnp.maximum(deg, 1.0)
    has = deg > 0
    C = c_ref[0]
    qblk = q_ref[0]
    Q1 = jnp.where(has, qblk[:, 0:FC], 0.0)
    Q2 = jnp.where(has, qblk[:, FC:2 * FC], 0.0)
    Qmn = jnp.where(has, qblk[:, 2 * FC:3 * FC], 0.0)
    Qmx = jnp.where(has, qblk[:, 3 * FC:4 * FC], 0.0)
    mean = (deg * C + Q1) / degc
    e2 = (deg * C * C + 2.0 * C * Q1 + Q2) / degc
    std = jnp.sqrt(jax.nn.relu(e2 - mean * mean) + EPS)
    mn = jnp.where(has, C + Qmn, 0.0)
    mx = jnp.where(has, C + Qmx, 0.0)
    amp = jnp.log(deg + 1.0) / AVG_LOG
    att = AVG_LOG / jnp.log(degc + 1.0)

    @pl.when(p == 0)
    def _init_acc():
        acc_ref[...] = jnp.dot(x_ref[...], wx_ref[...],
                               preferred_element_type=jnp.float32) + pb_ref[...]

    acc = acc_ref[...]
    parts = (mean, mn, mx, std)
    for k in range(4):
        a = parts[k]
        yp = jnp.dot(a, pw_ref[0, k], preferred_element_type=jnp.float32)
        ya = jnp.dot(a, pw_ref[0, 4 + k], preferred_element_type=jnp.float32)
        yt = jnp.dot(a, pw_ref[0, 8 + k], preferred_element_type=jnp.float32)
        acc += yp + amp * ya + att * yt
    acc_ref[...] = acc

    @pl.when(p == P - 1)
    def _fin():
        out = jnp.dot(acc, lw_ref[...], preferred_element_type=jnp.float32) + lb_ref[...]
        o_ref[...] = out

        @pl.when(i == 0)
        def _init_s():
            s1_ref[...] = jnp.zeros_like(s1_ref)
            s2_ref[...] = jnp.zeros_like(s2_ref)

        s1_ref[...] += jnp.sum(out, axis=0, keepdims=True)
        s2_ref[...] += jnp.sum(out * out, axis=0, keepdims=True)


def _post(x, C3, Q, deg2d, wx, pw, pb, lin_W, lin_b):
    return pl.pallas_call(
        _post_body,
        grid=(N // NB, P),
        in_specs=[
            pl.BlockSpec((NB, F), lambda i, p: (i, 0)),
            pl.BlockSpec((1, NB, FC), lambda i, p: (p, i, 0)),
            pl.BlockSpec((1, NB, 4 * FC), lambda i, p: (p, i, 0)),
            pl.BlockSpec((NB, 16), lambda i, p: (i, 0)),
            pl.BlockSpec((F, HID), lambda i, p: (0, 0)),
            pl.BlockSpec((1, 12, F, HID), lambda i, p: (p, 0, 0, 0)),
            pl.BlockSpec((1, HID), lambda i, p: (0, 0)),
            pl.BlockSpec((HID, HID), lambda i, p: (0, 0)),
            pl.BlockSpec((1, HID), lambda i, p: (0, 0)),
        ],
        out_specs=[
            pl.BlockSpec((NB, HID), lambda i, p: (i, 0)),
            pl.BlockSpec((1, HID), lambda i, p: (0, 0)),
            pl.BlockSpec((1, HID), lambda i, p: (0, 0)),
        ],
        out_shape=[
            jax.ShapeDtypeStruct((N, HID), jnp.float32),
            jax.ShapeDtypeStruct((1, HID), jnp.float32),
            jax.ShapeDtypeStruct((1, HID), jnp.float32),
        ],
        scratch_shapes=[pltpu.VMEM((NB, HID), jnp.float32)],
    )(x, C3, Q, deg2d, wx, pw, pb, lin_W, lin_b)


PAD = 8


def _final_body(o_ref, s1_ref, s2_ref, oneh_ref, bg_ref, bb_ref,
                w1_ref, b1_ref, w2_ref, b2_ref, w3_ref, b3_ref, out_ref,
                acc_ref):
    i = pl.program_id(0)
    mu = s1_ref[...] / N
    var = s2_ref[...] / N - mu * mu
    o = (o_ref[...] - mu) / jnp.sqrt(var + EPS) * bg_ref[...] + bb_ref[...]
    o = jax.nn.relu(o)

    @pl.when(i == 0)
    def _init():
        acc_ref[...] = jnp.zeros_like(acc_ref)

    acc_ref[...] += jax.lax.dot_general(
        oneh_ref[...], o, (((0,), (0,)), ((), ())),
        preferred_element_type=jnp.float32)

    @pl.when(i == pl.num_programs(0) - 1)
    def _fin():
        g = acc_ref[...]
        g = jax.nn.relu(jnp.dot(g, w1_ref[...], preferred_element_type=jnp.float32) + b1_ref[...])
        g = jax.nn.relu(jnp.dot(g, w2_ref[...], preferred_element_type=jnp.float32) + b2_ref[...])
        g = jnp.dot(g, w3_ref[...], preferred_element_type=jnp.float32) + b3_ref[...]
        col = jax.lax.broadcasted_iota(jnp.int32, (N_GRAPHS, PAD), 1)
        g = jnp.where(col < 2, g, -1e30)
        m = jnp.max(g, axis=-1, keepdims=True)
        s = jnp.log(jnp.sum(jnp.exp(g - m), axis=-1, keepdims=True))
        out_ref[...] = g - m - s


def _final(o, s1, s2, oneh, bn_gamma, bn_beta, mlp_W1, mlp_b1, mlp_W2, mlp_b2,
           w3p, b3p):
    return pl.pallas_call(
        _final_body,
        grid=(N // NB,),
        in_specs=[
            pl.BlockSpec((NB, HID), lambda i: (i, 0)),
            pl.BlockSpec((1, HID), lambda i: (0, 0)),
            pl.BlockSpec((1, HID), lambda i: (0, 0)),
            pl.BlockSpec((NB, N_GRAPHS), lambda i: (i, 0)),
            pl.BlockSpec((1, HID), lambda i: (0, 0)),
            pl.BlockSpec((1, HID), lambda i: (0, 0)),
            pl.BlockSpec((HID, HID // 2), lambda i: (0, 0)),
            pl.BlockSpec((1, HID // 2), lambda i: (0, 0)),
            pl.BlockSpec((HID // 2, HID // 4), lambda i: (0, 0)),
            pl.BlockSpec((1, HID // 4), lambda i: (0, 0)),
            pl.BlockSpec((HID // 4, PAD), lambda i: (0, 0)),
            pl.BlockSpec((1, PAD), lambda i: (0, 0)),
        ],
        out_specs=pl.BlockSpec((N_GRAPHS, PAD), lambda i: (0, 0)),
        out_shape=jax.ShapeDtypeStruct((N_GRAPHS, PAD), jnp.float32),
        scratch_shapes=[pltpu.VMEM((N_GRAPHS, HID), jnp.float32)],
    )(o, s1, s2, oneh, bn_gamma, bn_beta, mlp_W1, mlp_b1, mlp_W2, mlp_b2,
      w3p, b3p)


def kernel(x, edge_index, edge_attr, batch, edge_enc_W, edge_enc_b, pre_W,
           pre_b, post_W, post_b, lin_W, lin_b, bn_gamma, bn_beta, mlp_W1,
           mlp_b1, mlp_W2, mlp_b2, mlp_W3, mlp_b3):
    src, dst = edge_index[0], edge_index[1]
    W1 = pre_W[:, :F, :]
    W2 = pre_W[:, F:2 * F, :]
    W3 = pre_W[:, 2 * F:, :]
    v = jnp.einsum('f,tfo->to', edge_enc_W[0], W3).reshape(-1)
    c0 = (jnp.einsum('f,tfo->to', edge_enc_b, W3) + pre_b).reshape(-1)

    w13 = W1.transpose(1, 0, 2).reshape(F, TF).reshape(F, P, FC).transpose(1, 0, 2)
    w23 = W2.transpose(1, 0, 2).reshape(F, TF).reshape(F, P, FC).transpose(1, 0, 2)
    c03 = jnp.broadcast_to(c0.reshape(P, 1, FC), (P, 8, FC))
    C3, S3 = _compute_c_s(x, w13, w23, c03)
    s2d = S3.reshape(P * N, FC)

    rec = jnp.stack([src.astype(jnp.int32), dst.astype(jnp.int32),
                     jax.lax.bitcast_convert_type(edge_attr, jnp.int32)])
    v3 = v.reshape(P, FC)

    Qf, degq = _sc_edge(s2d, rec, v3)
    Q = Qf.reshape(P, EPAD, 4 * FC)
    deg2d = degq.reshape(EPAD, 16)

    wx = post_W[:, :F, :].transpose(1, 0, 2).reshape(F, T * F_OUT)
    eye_t = jnp.eye(T, dtype=jnp.float32)
    pw_parts = []
    for k in range(12):
        wk = post_W[:, (1 + k) * F:(2 + k) * F, :]
        pw_parts.append(jnp.einsum('pfo,pt->pfto', wk, eye_t).reshape(P, F, T * F_OUT))
    pw = jnp.stack(pw_parts, axis=1)
    pb = post_b.reshape(1, HID)

    o, s1, s2 = _post(x, C3, Q, deg2d, wx, pw, pb, lin_W,
                      lin_b.reshape(1, -1))

    oneh = (batch[:, None] == jnp.arange(N_GRAPHS, dtype=batch.dtype)[None, :]).astype(jnp.float32)
    w3p = jnp.zeros((HID // 4, PAD), jnp.float32).at[:, :2].set(mlp_W3)
    b3p = jnp.zeros((1, PAD), jnp.float32).at[:, :2].set(mlp_b3)
    outp = _final(o, s1, s2, oneh, bn_gamma.reshape(1, HID),
                  bn_beta.reshape(1, HID), mlp_W1, mlp_b1.reshape(1, -1),
                  mlp_W2, mlp_b2.reshape(1, -1), w3p, b3p)
    return outp[:, :2]


# speedup vs baseline: 1.1146x; 1.0019x over previous
import jax
import jax.numpy as jnp
import numpy as _np
from jax import lax
from jax.experimental import pallas as pl
from jax.experimental.pallas import tpu as pltpu
from jax.experimental.pallas import tpu_sc as plsc

N = 10000
E = 160000
F = 128
T = 8
F_OUT = 4
HID = 32
N_GRAPHS = 64
EPS = 1e-5
TF = T * F

_DEG_HIST = _np.array([0.0] * 8 + [625.0] * 16 + [0.0] * 9)
_bins = _np.arange(_DEG_HIST.shape[0], dtype=_np.float64)
AVG_LOG = float((_np.log(_bins + 1.0) * _DEG_HIST).sum() / _DEG_HIST.sum())

NB = 400
P = 8
FC = 128
NW = 32
NPW = 320
NH = 160
QCAP = 3456
SCHUNK = 1024
GCH = 64
EPAD = NW * NPW
BIG = 1e30


def _cs_body(x_ref, w1_ref, w2_ref, c0_ref, c_ref, s_ref):
    xb = x_ref[...]
    c_ref[0] = jnp.dot(xb, w1_ref[0], preferred_element_type=jnp.float32) + c0_ref[0, 0:1, :]
    s_ref[0] = jnp.dot(xb, w2_ref[0], preferred_element_type=jnp.float32)


def _compute_c_s(x, w13, w23, c03):
    return pl.pallas_call(
        _cs_body,
        grid=(N // NB, P),
        in_specs=[
            pl.BlockSpec((NB, F), lambda i, p: (i, 0)),
            pl.BlockSpec((1, F, FC), lambda i, p: (p, 0, 0)),
            pl.BlockSpec((1, F, FC), lambda i, p: (p, 0, 0)),
            pl.BlockSpec((1, 8, FC), lambda i, p: (p, 0, 0)),
        ],
        out_specs=[
            pl.BlockSpec((1, NB, FC), lambda i, p: (p, i, 0)),
            pl.BlockSpec((1, NB, FC), lambda i, p: (p, i, 0)),
        ],
        out_shape=[
            jax.ShapeDtypeStruct((P, N, FC), jnp.float32),
            jax.ShapeDtypeStruct((P, N, FC), jnp.float32),
        ],
    )(x, w13, w23, c03)


def _sc_edge_body(s2d, rec, v3, q_out, deg_out,
                  acc, dacc, qsrc, qdst, qea, recb, rows, gidx, vvb, sem):
    wid = lax.axis_index("c") * 16 + lax.axis_index("s")
    n0w = pl.multiple_of(wid * NPW, 16)

    def qz_body(i, _):
        z = jnp.zeros((16,), jnp.int32)
        qsrc[pl.ds(i * 16, 16)] = z
        qdst[pl.ds(i * 16, 16)] = jnp.full((16,), NH, jnp.int32)
        qea[pl.ds(i * 16, 16)] = z
        return 0
    lax.fori_loop(0, 2 * QCAP // 16, qz_body, 0)

    def scan_chunk(c, curs):
        base = pl.multiple_of(c * SCHUNK, 8)
        pltpu.sync_copy(rec.at[:, pl.ds(base, SCHUNK)], recb)

        def scan_vec(i, curs):
            c_lo, c_hi = curs
            sv = recb[0, pl.ds(i * 16, 16)]
            dv = recb[1, pl.ds(i * 16, 16)]
            ev = recb[2, pl.ds(i * 16, 16)]
            dl = dv - n0w
            m_lo = (dl >= 0) & (dl < NH)
            m_hi = (dl >= NH) & (dl < NPW)
            pos_lo = plsc.cumsum(m_lo.astype(jnp.int32))
            pos_hi = plsc.cumsum(m_hi.astype(jnp.int32))
            trash = jnp.full((16,), 2 * QCAP + 15, jnp.int32)
            idx_lo = jnp.where(m_lo, c_lo + pos_lo - 1, trash)
            idx_hi = jnp.where(m_hi, QCAP + c_hi + pos_hi - 1, trash)
            plsc.store_scatter(qsrc, [idx_lo], sv)
            plsc.store_scatter(qdst, [idx_lo], dl)
            plsc.store_scatter(qea, [idx_lo], ev)
            plsc.store_scatter(qsrc, [idx_hi], sv)
            plsc.store_scatter(qdst, [idx_hi], dl - NH)
            plsc.store_scatter(qea, [idx_hi], ev)
            c_lo = jnp.minimum(c_lo + pos_lo[15], QCAP - 16)
            c_hi = jnp.minimum(c_hi + pos_hi[15], QCAP - 16)
            return (c_lo, c_hi)

        return lax.fori_loop(0, SCHUNK // 16, scan_vec, curs)

    cnt_lo, cnt_hi = lax.fori_loop(0, E // SCHUNK, scan_chunk,
                                   (jnp.int32(0), jnp.int32(0)))
    padv_s = jnp.zeros((16,), jnp.int32)
    padv_d = jnp.full((16,), NH, jnp.int32)
    qsrc[pl.ds(cnt_lo, 16)] = padv_s
    qdst[pl.ds(cnt_lo, 16)] = padv_d
    qea[pl.ds(cnt_lo, 16)] = padv_s
    qsrc[pl.ds(QCAP + cnt_hi, 16)] = padv_s
    qdst[pl.ds(QCAP + cnt_hi, 16)] = padv_d
    qea[pl.ds(QCAP + cnt_hi, 16)] = padv_s
    ng_lo = lax.div(cnt_lo + 15, 16)
    ng_hi = lax.div(cnt_hi + 15, 16)

    zero = jnp.zeros((16,), jnp.float32)
    big = jnp.full((16,), BIG, jnp.float32)
    nbig = jnp.full((16,), -BIG, jnp.float32)
    one0 = (lax.iota(jnp.int32, 16) == 0).astype(jnp.float32)

    def pass_body(p, _):
        pltpu.sync_copy(v3.at[p], vvb)
        vv = [vvb[pl.ds(k * 16, 16)] for k in range(8)]
        pN = p * N

        def half_body(h, _):
            qb = h * QCAP
            ngrp = jnp.where(h == 0, ng_lo, ng_hi)
            n0 = n0w + h * NH

            def init_body(r, _):
                rb = r * (4 * FC)
                for k in range(8):
                    acc[pl.ds(rb + k * 16, 16)] = zero
                for k in range(8):
                    acc[pl.ds(rb + 128 + k * 16, 16)] = zero
                for k in range(8):
                    acc[pl.ds(rb + 256 + k * 16, 16)] = big
                for k in range(8):
                    acc[pl.ds(rb + 384 + k * 16, 16)] = nbig
                dacc[pl.ds(r * 16, 16)] = zero
                return 0
            lax.fori_loop(0, NH + 1, init_body, 0)

            nchunk = lax.div(ngrp + (GCH // 16 - 1), GCH // 16)

            def issue(c, par):
                gb = qb + c * GCH
                for i in range(GCH // 16):
                    gidx[par, pl.ds(i * 16, 16)] = (
                        qsrc[pl.ds(gb + i * 16, 16)] + pN)
                pltpu.async_copy(s2d.at[gidx.at[par]], rows.at[par],
                                 sem.at[par])

            @pl.when(nchunk > 0)
            def _prime():
                issue(jnp.int32(0), jnp.int32(0))

            def gchunk_body(c, _):
                par = lax.rem(c, 2)
                gb = qb + c * GCH
                pltpu.make_async_copy(s2d.at[gidx.at[par]], rows.at[par],
                                      sem.at[par]).wait()

                @pl.when(c + 1 < nchunk)
                def _next():
                    issue(c + 1, 1 - par)

                def grp_body(g, _):
                    dlv = qdst[pl.ds(gb + g * 16, 16)]
                    eav = plsc.bitcast(qea[pl.ds(gb + g * 16, 16)], jnp.float32)
                    for l in range(16):
                        loc = dlv[l]
                        ea = eav[l]
                        sb = loc * (4 * FC)
                        db = loc * 16
                        plsc.addupdate(dacc.at[pl.ds(db, 16)], one0)
                        for k in range(8):
                            rowv = rows[par, g * 16 + l, pl.ds(k * 16, 16)]
                            qv = rowv + ea * vv[k]
                            plsc.addupdate(acc.at[pl.ds(sb + k * 16, 16)], qv)
                            plsc.addupdate(acc.at[pl.ds(sb + 128 + k * 16, 16)], qv * qv)
                            o2 = sb + 256 + k * 16
                            acc[pl.ds(o2, 16)] = jnp.minimum(acc[pl.ds(o2, 16)], qv)
                            o3 = sb + 384 + k * 16
                            acc[pl.ds(o3, 16)] = jnp.maximum(acc[pl.ds(o3, 16)], qv)
                    return 0

                ngl = jnp.minimum(ngrp - c * (GCH // 16), GCH // 16)
                ngl = jnp.maximum(ngl, 0)
                lax.fori_loop(0, ngl, grp_body, 0)
                return 0

            lax.fori_loop(0, nchunk, gchunk_body, 0)

            pltpu.sync_copy(acc.at[pl.ds(0, NH * 4 * FC)],
                            q_out.at[p, pl.ds(n0 * (4 * FC), NH * 4 * FC)])

            @pl.when(p == 0)
            def _deg_out():
                pltpu.sync_copy(dacc.at[pl.ds(0, NH * 16)],
                                deg_out.at[pl.ds(n0 * 16, NH * 16)])
            return 0

        lax.fori_loop(0, 2, half_body, 0)
        return 0

    lax.fori_loop(0, P, pass_body, 0)


def _sc_edge(s2d, rec, v3):
    mesh = plsc.VectorSubcoreMesh(core_axis_name="c", subcore_axis_name="s")
    f = pl.kernel(
        _sc_edge_body,
        out_type=[
            jax.ShapeDtypeStruct((P, EPAD * 4 * FC), jnp.float32),
            jax.ShapeDtypeStruct((EPAD * 16,), jnp.float32),
        ],
        mesh=mesh,
        compiler_params=pltpu.CompilerParams(needs_layout_passes=False),
        scratch_types=[
            pltpu.VMEM(((NH + 1) * 4 * FC,), jnp.float32),
            pltpu.VMEM(((NH + 1) * 16,), jnp.float32),
            pltpu.VMEM((2 * QCAP + 16,), jnp.int32),
            pltpu.VMEM((2 * QCAP + 16,), jnp.int32),
            pltpu.VMEM((2 * QCAP + 16,), jnp.int32),
            pltpu.VMEM((3, SCHUNK), jnp.int32),
            pltpu.VMEM((2, GCH, FC), jnp.float32),
            pltpu.VMEM((2, GCH), jnp.int32),
            pltpu.VMEM((FC,), jnp.float32),
            pltpu.SemaphoreType.DMA((2,)),
        ],
    )
    return f(s2d, rec, v3)


def _post_body(x_ref, c_ref, q_ref, deg_ref, wx_ref, pw_ref, pb_ref,
               lw_ref, lb_ref, o_ref, s1_ref, s2_ref, acc_ref):
    i = pl.program_id(0)
    p = pl.program_id(1)
    deg = deg_ref[:, 0:1]
    degc = jnp.maximum(deg, 1.0)
    has = deg > 0
    C = c_ref[0]
    qblk = q_ref[0]
    Q1 = jnp.where(has, qblk[:, 0:FC], 0.0)
    Q2 = jnp.where(has, qblk[:, FC:2 * FC], 0.0)
    Qmn = jnp.where(has, qblk[:, 2 * FC:3 * FC], 0.0)
    Qmx = jnp.where(has, qblk[:, 3 * FC:4 * FC], 0.0)
    mean = (deg * C + Q1) / degc
    e2 = (deg * C * C + 2.0 * C * Q1 + Q2) / degc
    std = jnp.sqrt(jax.nn.relu(e2 - mean * mean) + EPS)
    mn = jnp.where(has, C + Qmn, 0.0)
    mx = jnp.where(has, C + Qmx, 0.0)
    amp = jnp.log(deg + 1.0) / AVG_LOG
    att = AVG_LOG / jnp.log(degc + 1.0)

    @pl.when(p == 0)
    def _init_acc():
        acc_ref[...] = jnp.dot(x_ref[...], wx_ref[...],
                               preferred_element_type=jnp.float32) + pb_ref[...]

    acc = acc_ref[...]
    parts = (mean, mn, mx, std)
    for k in range(4):
        a = parts[k]
        yp = jnp.dot(a, pw_ref[0, k], preferred_element_type=jnp.float32)
        ya = jnp.dot(a, pw_ref[0, 4 + k], preferred_element_type=jnp.float32)
        yt = jnp.dot(a, pw_ref[0, 8 + k], preferred_element_type=jnp.float32)
        acc += yp + amp * ya + att * yt
    acc_ref[...] = acc

    @pl.when(p == P - 1)
    def _fin():
        out = jnp.dot(acc, lw_ref[...], preferred_element_type=jnp.float32) + lb_ref[...]
        o_ref[...] = out

        @pl.when(i == 0)
        def _init_s():
            s1_ref[...] = jnp.zeros_like(s1_ref)
            s2_ref[...] = jnp.zeros_like(s2_ref)

        s1_ref[...] += jnp.sum(out, axis=0, keepdims=True)
        s2_ref[...] += jnp.sum(out * out, axis=0, keepdims=True)


def _post(x, C3, Q, deg2d, wx, pw, pb, lin_W, lin_b):
    return pl.pallas_call(
        _post_body,
        grid=(N // NB, P),
        in_specs=[
            pl.BlockSpec((NB, F), lambda i, p: (i, 0)),
            pl.BlockSpec((1, NB, FC), lambda i, p: (p, i, 0)),
            pl.BlockSpec((1, NB, 4 * FC), lambda i, p: (p, i, 0)),
            pl.BlockSpec((NB, 16), lambda i, p: (i, 0)),
            pl.BlockSpec((F, HID), lambda i, p: (0, 0)),
            pl.BlockSpec((1, 12, F, HID), lambda i, p: (p, 0, 0, 0)),
            pl.BlockSpec((1, HID), lambda i, p: (0, 0)),
            pl.BlockSpec((HID, HID), lambda i, p: (0, 0)),
            pl.BlockSpec((1, HID), lambda i, p: (0, 0)),
        ],
        out_specs=[
            pl.BlockSpec((NB, HID), lambda i, p: (i, 0)),
            pl.BlockSpec((1, HID), lambda i, p: (0, 0)),
            pl.BlockSpec((1, HID), lambda i, p: (0, 0)),
        ],
        out_shape=[
            jax.ShapeDtypeStruct((N, HID), jnp.float32),
            jax.ShapeDtypeStruct((1, HID), jnp.float32),
            jax.ShapeDtypeStruct((1, HID), jnp.float32),
        ],
        scratch_shapes=[pltpu.VMEM((NB, HID), jnp.float32)],
    )(x, C3, Q, deg2d, wx, pw, pb, lin_W, lin_b)


PAD = 8


def _final_body(o_ref, s1_ref, s2_ref, oneh_ref, bg_ref, bb_ref,
                w1_ref, b1_ref, w2_ref, b2_ref, w3_ref, b3_ref, out_ref,
                acc_ref):
    i = pl.program_id(0)
    mu = s1_ref[...] / N
    var = s2_ref[...] / N - mu * mu
    o = (o_ref[...] - mu) / jnp.sqrt(var + EPS) * bg_ref[...] + bb_ref[...]
    o = jax.nn.relu(o)

    @pl.when(i == 0)
    def _init():
        acc_ref[...] = jnp.zeros_like(acc_ref)

    acc_ref[...] += jax.lax.dot_general(
        oneh_ref[...], o, (((0,), (0,)), ((), ())),
        preferred_element_type=jnp.float32)

    @pl.when(i == pl.num_programs(0) - 1)
    def _fin():
        g = acc_ref[...]
        g = jax.nn.relu(jnp.dot(g, w1_ref[...], preferred_element_type=jnp.float32) + b1_ref[...])
        g = jax.nn.relu(jnp.dot(g, w2_ref[...], preferred_element_type=jnp.float32) + b2_ref[...])
        g = jnp.dot(g, w3_ref[...], preferred_element_type=jnp.float32) + b3_ref[...]
        col = jax.lax.broadcasted_iota(jnp.int32, (N_GRAPHS, PAD), 1)
        g = jnp.where(col < 2, g, -1e30)
        m = jnp.max(g, axis=-1, keepdims=True)
        s = jnp.log(jnp.sum(jnp.exp(g - m), axis=-1, keepdims=True))
        out_ref[...] = g - m - s


def _final(o, s1, s2, oneh, bn_gamma, bn_beta, mlp_W1, mlp_b1, mlp_W2, mlp_b2,
           w3p, b3p):
    return pl.pallas_call(
        _final_body,
        grid=(N // NB,),
        in_specs=[
            pl.BlockSpec((NB, HID), lambda i: (i, 0)),
            pl.BlockSpec((1, HID), lambda i: (0, 0)),
            pl.BlockSpec((1, HID), lambda i: (0, 0)),
            pl.BlockSpec((NB, N_GRAPHS), lambda i: (i, 0)),
            pl.BlockSpec((1, HID), lambda i: (0, 0)),
            pl.BlockSpec((1, HID), lambda i: (0, 0)),
            pl.BlockSpec((HID, HID // 2), lambda i: (0, 0)),
            pl.BlockSpec((1, HID // 2), lambda i: (0, 0)),
            pl.BlockSpec((HID // 2, HID // 4), lambda i: (0, 0)),
            pl.BlockSpec((1, HID // 4), lambda i: (0, 0)),
            pl.BlockSpec((HID // 4, PAD), lambda i: (0, 0)),
            pl.BlockSpec((1, PAD), lambda i: (0, 0)),
        ],
        out_specs=pl.BlockSpec((N_GRAPHS, PAD), lambda i: (0, 0)),
        out_shape=jax.ShapeDtypeStruct((N_GRAPHS, PAD), jnp.float32),
        scratch_shapes=[pltpu.VMEM((N_GRAPHS, HID), jnp.float32)],
    )(o, s1, s2, oneh, bn_gamma, bn_beta, mlp_W1, mlp_b1, mlp_W2, mlp_b2,
      w3p, b3p)


def kernel(x, edge_index, edge_attr, batch, edge_enc_W, edge_enc_b, pre_W,
           pre_b, post_W, post_b, lin_W, lin_b, bn_gamma, bn_beta, mlp_W1,
           mlp_b1, mlp_W2, mlp_b2, mlp_W3, mlp_b3):
    src, dst = edge_index[0], edge_index[1]
    W1 = pre_W[:, :F, :]
    W2 = pre_W[:, F:2 * F, :]
    W3 = pre_W[:, 2 * F:, :]
    v = jnp.einsum('f,tfo->to', edge_enc_W[0], W3).reshape(-1)
    c0 = (jnp.einsum('f,tfo->to', edge_enc_b, W3) + pre_b).reshape(-1)

    w13 = W1.transpose(1, 0, 2).reshape(F, TF).reshape(F, P, FC).transpose(1, 0, 2)
    w23 = W2.transpose(1, 0, 2).reshape(F, TF).reshape(F, P, FC).transpose(1, 0, 2)
    c03 = jnp.broadcast_to(c0.reshape(P, 1, FC), (P, 8, FC))
    C3, S3 = _compute_c_s(x, w13, w23, c03)
    s2d = S3.reshape(P * N, FC)

    rec = jnp.stack([src.astype(jnp.int32), dst.astype(jnp.int32),
                     jax.lax.bitcast_convert_type(edge_attr, jnp.int32)])
    v3 = v.reshape(P, FC)

    Qf, degq = _sc_edge(s2d, rec, v3)
    Q = Qf.reshape(P, EPAD, 4 * FC)
    deg2d = degq.reshape(EPAD, 16)

    wx = post_W[:, :F, :].transpose(1, 0, 2).reshape(F, T * F_OUT)
    eye_t = jnp.eye(T, dtype=jnp.float32)
    pw_parts = []
    for k in range(12):
        wk = post_W[:, (1 + k) * F:(2 + k) * F, :]
        pw_parts.append(jnp.einsum('pfo,pt->pfto', wk, eye_t).reshape(P, F, T * F_OUT))
    pw = jnp.stack(pw_parts, axis=1)
    pb = post_b.reshape(1, HID)

    o, s1, s2 = _post(x, C3, Q, deg2d, wx, pw, pb, lin_W,
                      lin_b.reshape(1, -1))

    oneh = (batch[:, None] == jnp.arange(N_GRAPHS, dtype=batch.dtype)[None, :]).astype(jnp.float32)
    w3p = jnp.zeros((HID // 4, PAD), jnp.float32).at[:, :2].set(mlp_W3)
    b3p = jnp.zeros((1, PAD), jnp.float32).at[:, :2].set(mlp_b3)
    outp = _final(o, s1, s2, oneh, bn_gamma.reshape(1, HID),
                  bn_beta.reshape(1, HID), mlp_W1, mlp_b1.reshape(1, -1),
                  mlp_W2, mlp_b2.reshape(1, -1), w3p, b3p)
    return outp[:, :2]
